# baseline (speedup 1.0000x reference)
import functools

import jax
import jax.numpy as jnp
from jax import lax
from jax.experimental import pallas as pl
from jax.experimental.pallas import tpu as pltpu
from jax.experimental.pallas import tpu_sc as plsc

B, S, D = 4096, 200, 64
DQ = D // 4
N_TOK = B * S
N_ROWS = 24 * 7 * 12

NC, NS = 2, 16
NW = NC * NS
TOK_PER_W = N_TOK // NW
CHUNK = 800
N_CHUNKS = TOK_PER_W // CHUNK

BLK_B = 64
N_HALF = BLK_B * S // 2


def _table_body(hour_ref, week_ref, month_ref, season_ref, w_ref, b_ref, ft_ref):
    w = w_ref[...]
    ht = jnp.dot(hour_ref[...], w[:, 0:DQ].T, preferred_element_type=jnp.float32)
    wt = jnp.dot(week_ref[...], w[:, DQ:2 * DQ].T, preferred_element_type=jnp.float32)
    mt = jnp.dot(month_ref[...], w[:, 2 * DQ:3 * DQ].T, preferred_element_type=jnp.float32)
    st = jnp.dot(season_ref[...], w[:, 3 * DQ:4 * DQ].T, preferred_element_type=jnp.float32)
    s_oh = (lax.broadcasted_iota(jnp.int32, (12, 4), 0) // 3
            == lax.broadcasted_iota(jnp.int32, (12, 4), 1)).astype(jnp.float32)
    mt2 = mt + jnp.dot(s_oh, st, preferred_element_type=jnp.float32) + b_ref[...][None, :]
    c_h = lax.broadcasted_iota(jnp.int32, (N_ROWS, 24), 0) // 84
    oh_h = (c_h == lax.broadcasted_iota(jnp.int32, (N_ROWS, 24), 1)).astype(jnp.float32)
    c_w = (lax.broadcasted_iota(jnp.int32, (N_ROWS, 7), 0) // 12) % 7
    oh_w = (c_w == lax.broadcasted_iota(jnp.int32, (N_ROWS, 7), 1)).astype(jnp.float32)
    c_m = lax.broadcasted_iota(jnp.int32, (N_ROWS, 12), 0) % 12
    oh_m = (c_m == lax.broadcasted_iota(jnp.int32, (N_ROWS, 12), 1)).astype(jnp.float32)
    ft_ref[...] = (jnp.dot(oh_h, ht, preferred_element_type=jnp.float32)
                   + jnp.dot(oh_w, wt, preferred_element_type=jnp.float32)
                   + jnp.dot(oh_m, mt2, preferred_element_type=jnp.float32))


def _build_table(hour_table, weekday_table, month_table, season_table, w, b):
    return pl.pallas_call(
        _table_body,
        out_shape=jax.ShapeDtypeStruct((N_ROWS, D), jnp.float32),
    )(hour_table, weekday_table, month_table, season_table, w, b)


def _idx_body(ts_ref, idx_ref):
    t = ts_ref[...]
    h = (t // 60) % 24
    wd = (t // 1440) % 7
    m = (t // 43200) % 12
    idx_ref[...] = h * 84 + wd * 12 + m


def _build_idx(timestamps):
    blk = 512
    return pl.pallas_call(
        _idx_body,
        grid=(B // blk,),
        in_specs=[pl.BlockSpec((blk, S), lambda i: (i, 0))],
        out_specs=pl.BlockSpec((blk, S), lambda i: (i, 0)),
        out_shape=jax.ShapeDtypeStruct((B, S), jnp.int32),
    )(timestamps)


def _sc_gather_body(ft_hbm, idx_hbm, out_hbm, idx_v, rows_v, sem):
    wid = lax.axis_index("s") * NC + lax.axis_index("c")
    tok0 = wid * TOK_PER_W

    def body(i, _):
        base = tok0 + i * CHUNK
        pltpu.sync_copy(idx_hbm.at[pl.ds(base, CHUNK)], idx_v)
        pltpu.async_copy(ft_hbm.at[idx_v], rows_v, sem).wait()
        pltpu.sync_copy(rows_v, out_hbm.at[pl.ds(base, CHUNK)])
        return 0

    lax.fori_loop(0, N_CHUNKS, body, 0)


@functools.cache
def _sc_gather():
    return functools.partial(
        pl.kernel,
        mesh=plsc.VectorSubcoreMesh(core_axis_name="c", subcore_axis_name="s"),
        out_type=jax.ShapeDtypeStruct((N_TOK, D), jnp.float32),
        scratch_types=[
            pltpu.VMEM((CHUNK,), jnp.int32),
            pltpu.VMEM((CHUNK, D), jnp.float32),
            pltpu.SemaphoreType.DMA,
        ],
        compiler_params=pltpu.CompilerParams(use_tc_tiling_on_sc=False),
    )(_sc_gather_body)


def _retile_body(x_ref, o_ref):
    x = x_ref[...]
    a = x[:, :D]
    c = x[:, D:]
    o_ref[...] = jnp.concatenate([a, c], axis=0).reshape(o_ref.shape)


def _retile(x2):
    return pl.pallas_call(
        _retile_body,
        grid=(B // BLK_B,),
        in_specs=[pl.BlockSpec((N_HALF, 128), lambda i: (i, 0))],
        out_specs=pl.BlockSpec((BLK_B, S, D), lambda i: (i, 0, 0)),
        out_shape=jax.ShapeDtypeStruct((B, S, D), jnp.float32),
    )(x2)


def kernel(timestamps, hour_table, weekday_table, month_table, season_table, W, b):
    ft = _build_table(hour_table, weekday_table, month_table, season_table, W, b)
    idx = _build_idx(timestamps).reshape(N_TOK)
    pidx = idx.reshape(B // BLK_B, 2, N_HALF).transpose(0, 2, 1).reshape(N_TOK)
    flat = _sc_gather()(ft, pidx)
    return _retile(flat.reshape(N_TOK // 2, 128))


# speedup vs baseline: 1.2530x; 1.2530x over previous
import functools

import jax
import jax.numpy as jnp
from jax import lax
from jax.experimental import pallas as pl
from jax.experimental.pallas import tpu as pltpu
from jax.experimental.pallas import tpu_sc as plsc

B, S, D = 4096, 200, 64
DQ = D // 4
N_TOK = B * S
N_ROWS = 24 * 7 * 12

NC, NS = 2, 16
NW = NC * NS
N_PAIR = N_TOK // 2
PAIR_PER_W = N_PAIR // NW
HALF = 400
N_CHUNKS = PAIR_PER_W // HALF

BLK_B = 64
N_HALF = BLK_B * S // 2


def _table_body(hour_ref, week_ref, month_ref, season_ref, w_ref, b_ref, ft_ref):
    w = w_ref[...]
    ht = jnp.dot(hour_ref[...], w[:, 0:DQ].T, preferred_element_type=jnp.float32)
    wt = jnp.dot(week_ref[...], w[:, DQ:2 * DQ].T, preferred_element_type=jnp.float32)
    mt = jnp.dot(month_ref[...], w[:, 2 * DQ:3 * DQ].T, preferred_element_type=jnp.float32)
    st = jnp.dot(season_ref[...], w[:, 3 * DQ:4 * DQ].T, preferred_element_type=jnp.float32)
    s_oh = (lax.broadcasted_iota(jnp.int32, (12, 4), 0) // 3
            == lax.broadcasted_iota(jnp.int32, (12, 4), 1)).astype(jnp.float32)
    mt2 = mt + jnp.dot(s_oh, st, preferred_element_type=jnp.float32) + b_ref[...][None, :]
    c_h = lax.broadcasted_iota(jnp.int32, (N_ROWS, 24), 0) // 84
    oh_h = (c_h == lax.broadcasted_iota(jnp.int32, (N_ROWS, 24), 1)).astype(jnp.float32)
    c_w = (lax.broadcasted_iota(jnp.int32, (N_ROWS, 7), 0) // 12) % 7
    oh_w = (c_w == lax.broadcasted_iota(jnp.int32, (N_ROWS, 7), 1)).astype(jnp.float32)
    c_m = lax.broadcasted_iota(jnp.int32, (N_ROWS, 12), 0) % 12
    oh_m = (c_m == lax.broadcasted_iota(jnp.int32, (N_ROWS, 12), 1)).astype(jnp.float32)
    ft_ref[...] = (jnp.dot(oh_h, ht, preferred_element_type=jnp.float32)
                   + jnp.dot(oh_w, wt, preferred_element_type=jnp.float32)
                   + jnp.dot(oh_m, mt2, preferred_element_type=jnp.float32))


def _build_table(hour_table, weekday_table, month_table, season_table, w, b):
    return pl.pallas_call(
        _table_body,
        out_shape=jax.ShapeDtypeStruct((N_ROWS, D), jnp.float32),
    )(hour_table, weekday_table, month_table, season_table, w, b)


def _idx_body(ts_ref, idx_ref):
    t = ts_ref[...]
    h = (t // 60) % 24
    wd = (t // 1440) % 7
    m = (t // 43200) % 12
    idx_ref[...] = h * 84 + wd * 12 + m


def _build_idx(timestamps):
    blk = 512
    return pl.pallas_call(
        _idx_body,
        grid=(B // blk,),
        in_specs=[pl.BlockSpec((blk, S), lambda i: (i, 0))],
        out_specs=pl.BlockSpec((blk, S), lambda i: (i, 0)),
        out_shape=jax.ShapeDtypeStruct((B, S), jnp.int32),
    )(timestamps)


def _sc_gather_body(ft_hbm, idxa_hbm, idxb_hbm, out_hbm, idxa_v, idxb_v, rows_a, rows_b, sem):
    wid = lax.axis_index("s") * NC + lax.axis_index("c")
    pair0 = wid * PAIR_PER_W

    def body(i, _):
        base = pair0 + i * HALF
        pltpu.sync_copy(idxa_hbm.at[pl.ds(base, HALF)], idxa_v)
        pltpu.sync_copy(idxb_hbm.at[pl.ds(base, HALF)], idxb_v)
        cp_a = pltpu.async_copy(ft_hbm.at[idxa_v], rows_a, sem)
        cp_b = pltpu.async_copy(ft_hbm.at[idxb_v], rows_b, sem)
        cp_a.wait()
        cp_b.wait()
        pltpu.sync_copy(rows_a, out_hbm.at[pl.ds(base, HALF), pl.ds(0, D)])
        pltpu.sync_copy(rows_b, out_hbm.at[pl.ds(base, HALF), pl.ds(D, D)])
        return 0

    lax.fori_loop(0, N_CHUNKS, body, 0)


@functools.cache
def _sc_gather():
    return functools.partial(
        pl.kernel,
        mesh=plsc.VectorSubcoreMesh(core_axis_name="c", subcore_axis_name="s"),
        out_type=jax.ShapeDtypeStruct((N_PAIR, 2 * D), jnp.float32),
        scratch_types=[
            pltpu.VMEM((HALF,), jnp.int32),
            pltpu.VMEM((HALF,), jnp.int32),
            pltpu.VMEM((HALF, D), jnp.float32),
            pltpu.VMEM((HALF, D), jnp.float32),
            pltpu.SemaphoreType.DMA,
        ],
        compiler_params=pltpu.CompilerParams(use_tc_tiling_on_sc=False),
    )(_sc_gather_body)


def _retile_body(x_ref, o_ref):
    x = x_ref[...]
    a = x[:, :D]
    c = x[:, D:]
    o_ref[...] = jnp.concatenate([a, c], axis=0).reshape(o_ref.shape)


def _retile(x2):
    return pl.pallas_call(
        _retile_body,
        grid=(B // BLK_B,),
        in_specs=[pl.BlockSpec((N_HALF, 128), lambda i: (i, 0))],
        out_specs=pl.BlockSpec((BLK_B, S, D), lambda i: (i, 0, 0)),
        out_shape=jax.ShapeDtypeStruct((B, S, D), jnp.float32),
    )(x2)


def kernel(timestamps, hour_table, weekday_table, month_table, season_table, W, b):
    ft = _build_table(hour_table, weekday_table, month_table, season_table, W, b)
    idx = _build_idx(timestamps).reshape(N_TOK)
    idx3 = idx.reshape(B // BLK_B, 2, N_HALF)
    idx_a = idx3[:, 0, :].reshape(N_PAIR)
    idx_b = idx3[:, 1, :].reshape(N_PAIR)
    flat2 = _sc_gather()(ft, idx_a, idx_b)
    return _retile(flat2)


# speedup vs baseline: 1.8410x; 1.4693x over previous
import functools

import jax
import jax.numpy as jnp
from jax import lax
from jax.experimental import pallas as pl
from jax.experimental.pallas import tpu as pltpu
from jax.experimental.pallas import tpu_sc as plsc

B, S, D = 4096, 200, 64
DQ = D // 4
N_TOK = B * S
N_ROWS = 24 * 7 * 12

NC, NS = 2, 16
NW = NC * NS
N_PAIR = N_TOK // 2
PAIR_PER_W = N_PAIR // NW
HALF = 400
N_CHUNKS = PAIR_PER_W // HALF

HB = B // 2
BB = 128


def _table_body(hour_ref, week_ref, month_ref, season_ref, w_ref, b_ref, ft_ref):
    w = w_ref[...]
    ht = jnp.dot(hour_ref[...], w[:, 0:DQ].T, preferred_element_type=jnp.float32)
    wt = jnp.dot(week_ref[...], w[:, DQ:2 * DQ].T, preferred_element_type=jnp.float32)
    mt = jnp.dot(month_ref[...], w[:, 2 * DQ:3 * DQ].T, preferred_element_type=jnp.float32)
    st = jnp.dot(season_ref[...], w[:, 3 * DQ:4 * DQ].T, preferred_element_type=jnp.float32)
    s_oh = (lax.broadcasted_iota(jnp.int32, (12, 4), 0) // 3
            == lax.broadcasted_iota(jnp.int32, (12, 4), 1)).astype(jnp.float32)
    mt2 = mt + jnp.dot(s_oh, st, preferred_element_type=jnp.float32) + b_ref[...][None, :]
    c_h = lax.broadcasted_iota(jnp.int32, (N_ROWS, 24), 0) // 84
    oh_h = (c_h == lax.broadcasted_iota(jnp.int32, (N_ROWS, 24), 1)).astype(jnp.float32)
    c_w = (lax.broadcasted_iota(jnp.int32, (N_ROWS, 7), 0) // 12) % 7
    oh_w = (c_w == lax.broadcasted_iota(jnp.int32, (N_ROWS, 7), 1)).astype(jnp.float32)
    c_m = lax.broadcasted_iota(jnp.int32, (N_ROWS, 12), 0) % 12
    oh_m = (c_m == lax.broadcasted_iota(jnp.int32, (N_ROWS, 12), 1)).astype(jnp.float32)
    ft_ref[...] = (jnp.dot(oh_h, ht, preferred_element_type=jnp.float32)
                   + jnp.dot(oh_w, wt, preferred_element_type=jnp.float32)
                   + jnp.dot(oh_m, mt2, preferred_element_type=jnp.float32))


def _build_table(hour_table, weekday_table, month_table, season_table, w, b):
    return pl.pallas_call(
        _table_body,
        out_shape=jax.ShapeDtypeStruct((N_ROWS, D), jnp.float32),
    )(hour_table, weekday_table, month_table, season_table, w, b)


def _idx_body(ts_ref, idx_ref):
    t = ts_ref[...]
    h = (t // 60) % 24
    wd = (t // 1440) % 7
    m = (t // 43200) % 12
    idx_ref[...] = h * 84 + wd * 12 + m


def _build_idx(timestamps):
    blk = 512
    return pl.pallas_call(
        _idx_body,
        grid=(B // blk,),
        in_specs=[pl.BlockSpec((blk, S), lambda i: (i, 0))],
        out_specs=pl.BlockSpec((blk, S), lambda i: (i, 0)),
        out_shape=jax.ShapeDtypeStruct((B, S), jnp.int32),
    )(timestamps)


def _sc_gather_body(ft_hbm, idxa_hbm, idxb_hbm, out_hbm, idxa_v, idxb_v, rows_a, rows_b, sem):
    wid = lax.axis_index("s") * NC + lax.axis_index("c")
    pair0 = wid * PAIR_PER_W

    def body(i, _):
        base = pair0 + i * HALF
        pltpu.sync_copy(idxa_hbm.at[pl.ds(base, HALF)], idxa_v)
        pltpu.sync_copy(idxb_hbm.at[pl.ds(base, HALF)], idxb_v)
        cp_a = pltpu.async_copy(ft_hbm.at[idxa_v], rows_a, sem)
        cp_b = pltpu.async_copy(ft_hbm.at[idxb_v], rows_b, sem)
        cp_a.wait()
        cp_b.wait()
        pltpu.sync_copy(rows_a, out_hbm.at[pl.ds(base, HALF), pl.ds(0, D)])
        pltpu.sync_copy(rows_b, out_hbm.at[pl.ds(base, HALF), pl.ds(D, D)])
        return 0

    lax.fori_loop(0, N_CHUNKS, body, 0)


@functools.cache
def _sc_gather():
    return functools.partial(
        pl.kernel,
        mesh=plsc.VectorSubcoreMesh(core_axis_name="c", subcore_axis_name="s"),
        out_type=jax.ShapeDtypeStruct((N_PAIR, 2 * D), jnp.float32),
        scratch_types=[
            pltpu.VMEM((HALF,), jnp.int32),
            pltpu.VMEM((HALF,), jnp.int32),
            pltpu.VMEM((HALF, D), jnp.float32),
            pltpu.VMEM((HALF, D), jnp.float32),
            pltpu.SemaphoreType.DMA,
        ],
        compiler_params=pltpu.CompilerParams(use_tc_tiling_on_sc=False),
    )(_sc_gather_body)


def _transpose_body(x_ref, o_ref):
    i = pl.program_id(0)
    x = x_ref[...]
    a = jnp.where(i < HB // BB, x[:, :, :D], x[:, :, D:])
    o_ref[...] = jnp.transpose(a, (0, 2, 1))


def _transpose(x3):
    return pl.pallas_call(
        _transpose_body,
        grid=(B // BB,),
        in_specs=[pl.BlockSpec((S, BB, 128), lambda i: (0, i % (HB // BB), 0))],
        out_specs=pl.BlockSpec((S, D, BB), lambda i: (0, 0, i)),
        out_shape=jax.ShapeDtypeStruct((S, D, B), jnp.float32),
    )(x3)


def kernel(timestamps, hour_table, weekday_table, month_table, season_table, W, b):
    ft = _build_table(hour_table, weekday_table, month_table, season_table, W, b)
    idx = _build_idx(timestamps)
    idx_a = idx[:HB, :].T.reshape(N_PAIR)
    idx_b = idx[HB:, :].T.reshape(N_PAIR)
    flat2 = _sc_gather()(ft, idx_a, idx_b)
    ot = _transpose(flat2.reshape(S, HB, 128))
    return jnp.transpose(ot, (2, 0, 1))
